# baseline (speedup 1.0000x reference)
import jax
import jax.numpy as jnp
from jax import lax
from jax.experimental import pallas as pl
from jax.experimental.pallas import tpu as pltpu
from jax.experimental.pallas import tpu_sc as plsc

N = 320000
S = 10000
D = 128
DUMMY = S
S_PAD = S + 8
NC = 2
NS = 16
NW = NC * NS
RPW = N // NW
BLK = 8
NBLK = N // BLK
BPW = RPW // BLK
CH = 200
CB = CH // BLK
NCH = RPW // CH
CO = 128
NV = D // 16
NB = 2 * NW
NEG_INF = float("-inf")

_mesh = plsc.VectorSubcoreMesh(
    core_axis_name="c", subcore_axis_name="s", num_cores=NC, num_subcores=NS
)


def _seg_body(m8_hbm, x2_hbm, b_hbm, out_hbm, bndr_hbm, bnds_hbm,
              mb0, mb1, raw0, raw1, bb0, bb1, bfb0, bfb1, blb0, blb1,
              dl0, dl1, stage, sidx, sidxp, bndr_v, bnds_v, accb,
              semm0, semm1, semr0, semr1, semb0, semb1, sems, semo):
    wid = lax.axis_index("s") * NC + lax.axis_index("c")
    r0 = wid * RPW
    blk0 = wid * BPW
    iota = lax.iota(jnp.int32, 16)
    zeros16 = jnp.zeros((16,), jnp.int32)

    def issue_b(k, bb, semb):
        pltpu.async_copy(
            b_hbm.at[pl.ds(r0 + k * CH, CH)], bb.at[pl.ds(0, CH)], semb
        )

    def wait_b(k, bb, semb):
        pltpu.make_async_copy(
            b_hbm.at[pl.ds(r0 + k * CH, CH)], bb.at[pl.ds(0, CH)], semb
        ).wait()

    def issue_m(k, mb, semm):
        pltpu.async_copy(
            m8_hbm.at[pl.ds(blk0 + k * CB, CB)], mb, semm
        )

    def wait_m(k, mb, semm):
        pltpu.make_async_copy(
            m8_hbm.at[pl.ds(blk0 + k * CB, CB)], mb, semm
        ).wait()

    def prep(k, bb, bfb, blb, dl, raw, semr):
        base = blk0 + k * CB
        idx0 = iota * BLK
        bf0 = plsc.load_gather(bb, [idx0])
        bl0 = plsc.load_gather(bb, [idx0 + (BLK - 1)])
        idx1 = 16 * BLK + iota * BLK
        bf1 = plsc.load_gather(bb, [idx1])
        bl1 = plsc.load_gather(bb, [idx1 + (BLK - 1)])
        bfb[pl.ds(0, 16)] = bf0
        bfb[pl.ds(16, 16)] = bf1
        blb[pl.ds(0, 16)] = bl0
        blb[pl.ds(16, 16)] = bl1
        d0 = bf0 != bl0
        d1 = jnp.logical_and(bf1 != bl1, iota < CB - 16)
        dl[pl.ds(0, 16)] = zeros16
        dl[pl.ds(16, 16)] = zeros16
        n0 = plsc.all_reduce_population_count(d0)[0]
        n1 = plsc.all_reduce_population_count(d1)[0]
        pos0 = plsc.cumsum(d0.astype(jnp.int32)) - 1
        pos1 = plsc.cumsum(d1.astype(jnp.int32)) - 1 + n0
        plsc.store_scatter(dl, [pos0], base + iota, mask=d0)
        plsc.store_scatter(dl, [pos1], base + 16 + iota, mask=d1)
        nd = n0 + n1
        nq = (nd + 7) // 8
        for q in range(4):
            @pl.when(q < nq)
            def _():
                pltpu.async_copy(
                    x2_hbm.at[dl.at[pl.ds(8 * q, 8)]],
                    raw.at[pl.ds(8 * q, 8)],
                    semr,
                )
        return nq

    def wait_raw(nq, dl, raw, semr):
        for q in range(4):
            @pl.when(q < nq)
            def _():
                pltpu.make_async_copy(
                    x2_hbm.at[dl.at[pl.ds(8 * q, 8)]],
                    raw.at[pl.ds(8 * q, 8)],
                    semr,
                ).wait()

    def do_flush():
        for v in range(CO // 16):
            sidx[pl.ds(16 * v, 16)] = sidxp[pl.ds(16 * v, 16)]
        pltpu.async_copy(stage, out_hbm.at[sidx], sems).wait()

    def close_seg(cur, cnt, fd):
        def real(cnt_, fd_):
            def first_fn(cnt3):
                for j in range(NV):
                    bndr_v[0, pl.ds(16 * j, 16)] = accb[pl.ds(16 * j, 16)]
                bnds_v[0, :] = jnp.broadcast_to(cur, (16,))
                return cnt3

            def interior(cnt3):
                for j in range(NV):
                    stage[cnt3, pl.ds(16 * j, 16)] = accb[pl.ds(16 * j, 16)]
                sidxp[pl.ds(cnt3, 16)] = jnp.broadcast_to(cur, (16,))
                cnt4 = cnt3 + 1

                def flush():
                    do_flush()
                    return jnp.int32(0)

                return lax.cond(cnt4 == CO, flush, lambda: cnt4)

            cnt_o = lax.cond(fd_ == 1, interior, first_fn, cnt_)
            return cnt_o, jnp.int32(1)

        return lax.cond(cur >= 0, real, lambda a, b_: (a, b_), cnt, fd)

    def process(mb, raw, bb, bfb, blb, carry):
        def blk_body(blk, c):
            cur, cnt, fd, dptr = c
            bf = bfb[pl.ds(blk, 16)][0]
            bl = blb[pl.ds(blk, 16)][0]

            def clean_fn(cur_, cnt_, fd_, dptr_):
                ch = bf != cur_
                cnt_n, fd_n = lax.cond(
                    ch, lambda a, b_: close_seg(cur_, a, b_),
                    lambda a, b_: (a, b_), cnt_, fd_
                )
                for j in range(NV):
                    a = accb[pl.ds(16 * j, 16)]
                    m = mb[blk, pl.ds(16 * j, 16)]
                    accb[pl.ds(16 * j, 16)] = jnp.where(
                        ch, m, jnp.maximum(a, m)
                    )
                return (bf, cnt_n, fd_n, dptr_)

            def dirty_fn(cur_, cnt_, fd_, dptr_):
                def row(r, cc):
                    cur2, cnt2, fd2 = cc
                    s = bb[pl.ds(BLK * blk + r, 16)][0]
                    ch = s != cur2
                    cnt3, fd3 = lax.cond(
                        ch, lambda a, b_: close_seg(cur2, a, b_),
                        lambda a, b_: (a, b_), cnt2, fd2
                    )
                    for j in range(NV):
                        a = accb[pl.ds(16 * j, 16)]
                        xv = raw[dptr_, pl.ds(D * r + 16 * j, 16)]
                        accb[pl.ds(16 * j, 16)] = jnp.where(
                            ch, xv, jnp.maximum(a, xv)
                        )
                    return (s, cnt3, fd3)

                cur_n, cnt_n, fd_n = lax.fori_loop(
                    0, BLK, row, (cur_, cnt_, fd_)
                )
                return (cur_n, cnt_n, fd_n, dptr_ + 1)

            return lax.cond(bf == bl, clean_fn, dirty_fn, cur, cnt, fd, dptr)

        cur, cnt, fd, _ = lax.fori_loop(
            0, CB, blk_body, (carry[0], carry[1], carry[2], jnp.int32(0))
        )
        return (cur, cnt, fd)

    neg = jnp.full((16,), NEG_INF, jnp.float32)
    for j in range(NV):
        accb[pl.ds(16 * j, 16)] = neg
    issue_b(0, bb0, semb0)
    issue_b(1, bb1, semb1)
    wait_b(0, bb0, semb0)
    nq0_init = prep(0, bb0, bfb0, blb0, dl0, raw0, semr0)
    issue_m(0, mb0, semm0)
    carry0 = (jnp.int32(-1), jnp.int32(0), jnp.int32(0), nq0_init)
    npair = NCH // 2

    def pair(p, carry):
        cur, cnt, fd, nq0 = carry
        wait_b(2 * p + 1, bb1, semb1)
        nq1 = prep(2 * p + 1, bb1, bfb1, blb1, dl1, raw1, semr1)
        issue_m(2 * p + 1, mb1, semm1)

        wait_raw(nq0, dl0, raw0, semr0)
        wait_m(2 * p, mb0, semm0)
        c = process(mb0, raw0, bb0, bfb0, blb0, (cur, cnt, fd))

        @pl.when(p < npair - 1)
        def _():
            issue_b(2 * p + 2, bb0, semb0)

        def prep_even():
            wait_b(2 * p + 2, bb0, semb0)
            nq = prep(2 * p + 2, bb0, bfb0, blb0, dl0, raw0, semr0)
            issue_m(2 * p + 2, mb0, semm0)
            return nq

        nq0_n = lax.cond(p < npair - 1, prep_even, lambda: jnp.int32(0))

        wait_raw(nq1, dl1, raw1, semr1)
        wait_m(2 * p + 1, mb1, semm1)
        c = process(mb1, raw1, bb1, bfb1, blb1, c)

        @pl.when(p < npair - 1)
        def _():
            issue_b(2 * p + 3, bb1, semb1)

        return (c[0], c[1], c[2], nq0_n)

    carry = lax.fori_loop(0, npair, pair, carry0)
    cur, cnt, fd = carry[0], carry[1], carry[2]

    for j in range(NV):
        bndr_v[1, pl.ds(16 * j, 16)] = accb[pl.ds(16 * j, 16)]
    bnds_v[1, :] = jnp.broadcast_to(cur, (16,))

    @pl.when(fd == 0)
    def _():
        for j in range(NV):
            bndr_v[0, pl.ds(16 * j, 16)] = accb[pl.ds(16 * j, 16)]
        bnds_v[0, :] = jnp.broadcast_to(cur, (16,))

    for v in range(CO // 16):
        old = sidxp[pl.ds(16 * v, 16)]
        sidxp[pl.ds(16 * v, 16)] = jnp.where(
            iota + 16 * v >= cnt, jnp.int32(DUMMY), old
        )
    do_flush()

    pltpu.async_copy(bndr_v, bndr_hbm.at[pl.ds(2 * wid, 2)], semo).wait()
    pltpu.async_copy(bnds_v, bnds_hbm.at[pl.ds(2 * wid, 2)], semo).wait()


_seg_call = pl.kernel(
    _seg_body,
    out_type=(
        jax.ShapeDtypeStruct((NB, D), jnp.float32),
        jax.ShapeDtypeStruct((NB, 16), jnp.int32),
    ),
    mesh=_mesh,
    compiler_params=pltpu.CompilerParams(
        use_tc_tiling_on_sc=False, needs_layout_passes=False
    ),
    scratch_types=[
        pltpu.VMEM((CB, D), jnp.float32),
        pltpu.VMEM((CB, D), jnp.float32),
        pltpu.VMEM((32, BLK * D), jnp.float32),
        pltpu.VMEM((32, BLK * D), jnp.float32),
        pltpu.VMEM((264,), jnp.int32),
        pltpu.VMEM((264,), jnp.int32),
        pltpu.VMEM((48,), jnp.int32),
        pltpu.VMEM((48,), jnp.int32),
        pltpu.VMEM((48,), jnp.int32),
        pltpu.VMEM((48,), jnp.int32),
        pltpu.VMEM((32,), jnp.int32),
        pltpu.VMEM((32,), jnp.int32),
        pltpu.VMEM((CO, D), jnp.float32),
        pltpu.VMEM((CO,), jnp.int32),
        pltpu.VMEM((CO + 16,), jnp.int32),
        pltpu.VMEM((2, D), jnp.float32),
        pltpu.VMEM((2, 16), jnp.int32),
        pltpu.VMEM((D,), jnp.float32),
        pltpu.SemaphoreType.DMA,
        pltpu.SemaphoreType.DMA,
        pltpu.SemaphoreType.DMA,
        pltpu.SemaphoreType.DMA,
        pltpu.SemaphoreType.DMA,
        pltpu.SemaphoreType.DMA,
        pltpu.SemaphoreType.DMA,
        pltpu.SemaphoreType.DMA,
    ],
)


def _comb_body(bndr_hbm, bnds_hbm, out_hbm, br, bs, stage2, sidx2, sidx2p,
               sem1, sem2):
    wid = lax.axis_index("s") * NC + lax.axis_index("c")

    @pl.when(wid == 0)
    def _():
        pltpu.async_copy(bndr_hbm, br, sem1).wait()
        pltpu.async_copy(bnds_hbm, bs, sem1).wait()

        acc0 = tuple(br[0, pl.ds(16 * j, 16)] for j in range(NV))
        carry0 = (bs[0, pl.ds(0, 16)][0], jnp.int32(0)) + acc0

        def rec(r, c):
            cur, cnt = c[0], c[1]
            acc = c[2:]
            s = bs[r, pl.ds(0, 16)][0]
            ch = s != cur

            def close(cnt_):
                for j in range(NV):
                    stage2[cnt_, pl.ds(16 * j, 16)] = acc[j]
                sidx2p[pl.ds(cnt_, 16)] = jnp.broadcast_to(cur, (16,))
                return cnt_ + 1

            cnt_n = lax.cond(ch, close, lambda a: a, cnt)
            rv = [br[r, pl.ds(16 * j, 16)] for j in range(NV)]
            acc_n = [
                jnp.where(ch, rv[j], jnp.maximum(acc[j], rv[j]))
                for j in range(NV)
            ]
            return (s, cnt_n) + tuple(acc_n)

        carry = lax.fori_loop(1, NB, rec, carry0)
        cur, cnt = carry[0], carry[1]
        acc = carry[2:]
        for j in range(NV):
            stage2[cnt, pl.ds(16 * j, 16)] = acc[j]
        sidx2p[pl.ds(cnt, 16)] = jnp.broadcast_to(cur, (16,))
        cnt = cnt + 1

        lanes = lax.iota(jnp.int32, 16)
        for v in range(NB // 16):
            old = sidx2p[pl.ds(16 * v, 16)]
            sidx2p[pl.ds(16 * v, 16)] = jnp.where(
                lanes + 16 * v >= cnt, jnp.int32(DUMMY), old
            )
            sidx2[pl.ds(16 * v, 16)] = sidx2p[pl.ds(16 * v, 16)]
        pltpu.async_copy(stage2, out_hbm.at[sidx2], sem2).wait()


_comb_call = pl.kernel(
    _comb_body,
    out_type=(),
    mesh=_mesh,
    compiler_params=pltpu.CompilerParams(
        use_tc_tiling_on_sc=False, needs_layout_passes=False
    ),
    scratch_types=[
        pltpu.VMEM((NB, D), jnp.float32),
        pltpu.VMEM((NB, 16), jnp.int32),
        pltpu.VMEM((NB, D), jnp.float32),
        pltpu.VMEM((NB,), jnp.int32),
        pltpu.VMEM((NB + 16,), jnp.int32),
        pltpu.SemaphoreType.DMA,
        pltpu.SemaphoreType.DMA,
    ],
)

MBS = 3200


def _m8_body(x_ref, o_ref):
    xr = x_ref[...].reshape(MBS // BLK, BLK, D)
    o_ref[...] = jnp.max(xr, axis=1)


def _m8(x):
    return pl.pallas_call(
        _m8_body,
        grid=(N // MBS,),
        in_specs=[pl.BlockSpec((MBS, D), lambda i: (i, 0))],
        out_specs=pl.BlockSpec((MBS // BLK, D), lambda i: (i, 0)),
        out_shape=jax.ShapeDtypeStruct((NBLK, D), jnp.float32),
    )(x)


BS = 1000


def _mlp_body(xg_ref, xo_ref, w_ref, b_ref, o_ref):
    xg = xg_ref[...]
    xo = xo_ref[...]
    w = w_ref[...]
    h = lax.dot_general(xg, w[:, :D], (((1,), (1,)), ((), ())),
                        preferred_element_type=jnp.float32)
    h = h + lax.dot_general(xo, w[:, D:], (((1,), (1,)), ((), ())),
                            preferred_element_type=jnp.float32)
    h = h + b_ref[...]
    h = jnp.where(h >= 0, h, 0.01 * h)
    o_ref[...] = h + xo


def _mlp(xg, xg_old, W, b2):
    return pl.pallas_call(
        _mlp_body,
        grid=(S // BS,),
        in_specs=[
            pl.BlockSpec((BS, D), lambda i: (i, 0)),
            pl.BlockSpec((BS, D), lambda i: (i, 0)),
            pl.BlockSpec((D, 2 * D), lambda i: (0, 0)),
            pl.BlockSpec((1, D), lambda i: (0, 0)),
        ],
        out_specs=pl.BlockSpec((BS, D), lambda i: (i, 0)),
        out_shape=jax.ShapeDtypeStruct((S, D), jnp.float32),
    )(xg, xg_old, W, b2)


def kernel(xg_old, x, batch, W, b):
    batch = batch.astype(jnp.int32)
    m8 = _m8(x)
    x2d = x.reshape(NBLK, BLK * D)
    out_ref = jax.new_ref(jnp.full((S_PAD, D), NEG_INF, dtype=jnp.float32))
    bndr, bnds = _seg_call(m8, x2d, batch, out_ref)
    _comb_call(bndr, bnds, out_ref)
    xg = out_ref[...][:S]
    return _mlp(xg, xg_old, W, b.reshape(1, D))


# speedup vs baseline: 1.6879x; 1.6879x over previous
import jax
import jax.numpy as jnp
from jax import lax
from jax.experimental import pallas as pl
from jax.experimental.pallas import tpu as pltpu
from jax.experimental.pallas import tpu_sc as plsc

N = 320000
S = 10000
D = 128
DUMMY = S
S_PAD = S + 8
NC = 2
NS = 16
NW = NC * NS
RPW = N // NW
CH = 400
NCH = RPW // CH
NG = CH // 16
CO = 128
NV = D // 16
NB = 2 * NW
NEG_INF = float("-inf")

_mesh = plsc.VectorSubcoreMesh(
    core_axis_name="c", subcore_axis_name="s", num_cores=NC, num_subcores=NS
)


def _seg_body(x_hbm, b_hbm, out_hbm, bndr_hbm, bnds_hbm,
              xb0, xb1, bb0, bb1, stage, sidx, sidxp, bndr_v, bnds_v, accb,
              semx0, semx1, semb0, semb1, sems, semo):
    wid = lax.axis_index("s") * NC + lax.axis_index("c")
    r0 = wid * RPW

    def issue(k, xb, bb, semx, semb):
        base = r0 + k * CH
        pltpu.async_copy(x_hbm.at[pl.ds(base, CH)], xb, semx)
        pltpu.async_copy(b_hbm.at[pl.ds(base, CH)], bb.at[pl.ds(0, CH)], semb)

    def wait(k, xb, bb, semx, semb):
        base = r0 + k * CH
        pltpu.make_async_copy(x_hbm.at[pl.ds(base, CH)], xb, semx).wait()
        pltpu.make_async_copy(
            b_hbm.at[pl.ds(base, CH)], bb.at[pl.ds(0, CH)], semb
        ).wait()

    issue(0, xb0, bb0, semx0, semb0)
    issue(1, xb1, bb1, semx1, semb1)

    def do_flush():
        for v in range(CO // 16):
            sidx[pl.ds(16 * v, 16)] = sidxp[pl.ds(16 * v, 16)]
        pltpu.async_copy(stage, out_hbm.at[sidx], sems).wait()

    def run_chunk(xb, bb, carry):
        def group(g, c):
            bvec = bb[pl.ds(16 * g, 16)]
            first, last = bvec[0], bvec[15]
            clean = jnp.logical_and(first == c[0], first == last)

            def fast(c):
                acc = [accb[pl.ds(16 * j, 16)] for j in range(NV)]
                for i in range(16):
                    for j in range(NV):
                        acc[j] = jnp.maximum(
                            acc[j], xb[16 * g + i, pl.ds(16 * j, 16)]
                        )
                for j in range(NV):
                    accb[pl.ds(16 * j, 16)] = acc[j]
                return c

            def slow(c):
                acc = [accb[pl.ds(16 * j, 16)] for j in range(NV)]
                for i in range(16):
                    cur, cnt, fd = c
                    s = bvec[i]
                    ch = s != cur
                    acc_now = list(acc)

                    def on_change(cnt_, fd_):
                        def real(cnt__, fd__):
                            def first_fn(cnt3):
                                for j in range(NV):
                                    bndr_v[0, pl.ds(16 * j, 16)] = acc_now[j]
                                bnds_v[0, :] = jnp.broadcast_to(cur, (16,))
                                return cnt3

                            def interior(cnt3):
                                for j in range(NV):
                                    stage[cnt3, pl.ds(16 * j, 16)] = acc_now[j]
                                sidxp[pl.ds(cnt3, 16)] = jnp.broadcast_to(
                                    cur, (16,)
                                )
                                cnt4 = cnt3 + 1

                                def flush():
                                    do_flush()
                                    return jnp.int32(0)

                                return lax.cond(
                                    cnt4 == CO, flush, lambda: cnt4
                                )

                            cnt_o = lax.cond(
                                fd__ == 1, interior, first_fn, cnt__
                            )
                            return cnt_o, jnp.int32(1)

                        return lax.cond(
                            cur >= 0, real, lambda a, b_: (a, b_), cnt_, fd_
                        )

                    cnt_n, fd_n = lax.cond(
                        ch, on_change, lambda a, b_: (a, b_), cnt, fd
                    )
                    for j in range(NV):
                        xv = xb[16 * g + i, pl.ds(16 * j, 16)]
                        acc[j] = jnp.where(
                            ch, xv, jnp.maximum(acc_now[j], xv)
                        )
                    c = (s, cnt_n, fd_n)
                for j in range(NV):
                    accb[pl.ds(16 * j, 16)] = acc[j]
                return c

            return lax.cond(clean, fast, slow, c)

        return lax.fori_loop(0, NG, group, carry)

    neg = jnp.full((16,), NEG_INF, jnp.float32)
    for j in range(NV):
        accb[pl.ds(16 * j, 16)] = neg
    carry0 = (jnp.int32(-1), jnp.int32(0), jnp.int32(0))
    npair = NCH // 2

    def pair(p, carry):
        wait(2 * p, xb0, bb0, semx0, semb0)
        carry = run_chunk(xb0, bb0, carry)
        issue(2 * p + 2, xb0, bb0, semx0, semb0)

        wait(2 * p + 1, xb1, bb1, semx1, semb1)
        carry = run_chunk(xb1, bb1, carry)

        @pl.when(p < npair - 1)
        def _():
            issue(2 * p + 3, xb1, bb1, semx1, semb1)

        return carry

    carry = lax.fori_loop(0, npair, pair, carry0)
    wait(NCH - 1, xb0, bb0, semx0, semb0)
    carry = run_chunk(xb0, bb0, carry)

    cur, cnt, fd = carry[0], carry[1], carry[2]
    acc = [accb[pl.ds(16 * j, 16)] for j in range(NV)]

    for j in range(NV):
        bndr_v[1, pl.ds(16 * j, 16)] = acc[j]
    bnds_v[1, :] = jnp.broadcast_to(cur, (16,))

    @pl.when(fd == 0)
    def _():
        for j in range(NV):
            bndr_v[0, pl.ds(16 * j, 16)] = acc[j]
        bnds_v[0, :] = jnp.broadcast_to(cur, (16,))

    lanes = lax.iota(jnp.int32, 16)
    for v in range(CO // 16):
        old = sidxp[pl.ds(16 * v, 16)]
        sidxp[pl.ds(16 * v, 16)] = jnp.where(
            lanes + 16 * v >= cnt, jnp.int32(DUMMY), old
        )
    do_flush()

    pltpu.async_copy(bndr_v, bndr_hbm.at[pl.ds(2 * wid, 2)], semo).wait()
    pltpu.async_copy(bnds_v, bnds_hbm.at[pl.ds(2 * wid, 2)], semo).wait()


_seg_call = pl.kernel(
    _seg_body,
    out_type=(
        jax.ShapeDtypeStruct((NB, D), jnp.float32),
        jax.ShapeDtypeStruct((NB, 16), jnp.int32),
    ),
    mesh=_mesh,
    compiler_params=pltpu.CompilerParams(
        use_tc_tiling_on_sc=False, needs_layout_passes=False
    ),
    scratch_types=[
        pltpu.VMEM((CH, D), jnp.float32),
        pltpu.VMEM((CH, D), jnp.float32),
        pltpu.VMEM((CH + 16,), jnp.int32),
        pltpu.VMEM((CH + 16,), jnp.int32),
        pltpu.VMEM((CO, D), jnp.float32),
        pltpu.VMEM((CO,), jnp.int32),
        pltpu.VMEM((CO + 16,), jnp.int32),
        pltpu.VMEM((2, D), jnp.float32),
        pltpu.VMEM((2, 16), jnp.int32),
        pltpu.VMEM((D,), jnp.float32),
        pltpu.SemaphoreType.DMA,
        pltpu.SemaphoreType.DMA,
        pltpu.SemaphoreType.DMA,
        pltpu.SemaphoreType.DMA,
        pltpu.SemaphoreType.DMA,
        pltpu.SemaphoreType.DMA,
    ],
)


def _comb_body(bndr_hbm, bnds_hbm, out_hbm, br, bs, stage2, sidx2, sidx2p,
               sem1, sem2):
    wid = lax.axis_index("s") * NC + lax.axis_index("c")

    @pl.when(wid == 0)
    def _():
        pltpu.async_copy(bndr_hbm, br, sem1).wait()
        pltpu.async_copy(bnds_hbm, bs, sem1).wait()

        acc0 = tuple(br[0, pl.ds(16 * j, 16)] for j in range(NV))
        carry0 = (bs[0, pl.ds(0, 16)][0], jnp.int32(0)) + acc0

        def rec(r, c):
            cur, cnt = c[0], c[1]
            acc = c[2:]
            s = bs[r, pl.ds(0, 16)][0]
            ch = s != cur

            def close(cnt_):
                for j in range(NV):
                    stage2[cnt_, pl.ds(16 * j, 16)] = acc[j]
                sidx2p[pl.ds(cnt_, 16)] = jnp.broadcast_to(cur, (16,))
                return cnt_ + 1

            cnt_n = lax.cond(ch, close, lambda a: a, cnt)
            rv = [br[r, pl.ds(16 * j, 16)] for j in range(NV)]
            acc_n = [
                jnp.where(ch, rv[j], jnp.maximum(acc[j], rv[j]))
                for j in range(NV)
            ]
            return (s, cnt_n) + tuple(acc_n)

        carry = lax.fori_loop(1, NB, rec, carry0)
        cur, cnt = carry[0], carry[1]
        acc = carry[2:]
        for j in range(NV):
            stage2[cnt, pl.ds(16 * j, 16)] = acc[j]
        sidx2p[pl.ds(cnt, 16)] = jnp.broadcast_to(cur, (16,))
        cnt = cnt + 1

        lanes = lax.iota(jnp.int32, 16)
        for v in range(NB // 16):
            old = sidx2p[pl.ds(16 * v, 16)]
            sidx2p[pl.ds(16 * v, 16)] = jnp.where(
                lanes + 16 * v >= cnt, jnp.int32(DUMMY), old
            )
            sidx2[pl.ds(16 * v, 16)] = sidx2p[pl.ds(16 * v, 16)]
        pltpu.async_copy(stage2, out_hbm.at[sidx2], sem2).wait()


_comb_call = pl.kernel(
    _comb_body,
    out_type=(),
    mesh=_mesh,
    compiler_params=pltpu.CompilerParams(
        use_tc_tiling_on_sc=False, needs_layout_passes=False
    ),
    scratch_types=[
        pltpu.VMEM((NB, D), jnp.float32),
        pltpu.VMEM((NB, 16), jnp.int32),
        pltpu.VMEM((NB, D), jnp.float32),
        pltpu.VMEM((NB,), jnp.int32),
        pltpu.VMEM((NB + 16,), jnp.int32),
        pltpu.SemaphoreType.DMA,
        pltpu.SemaphoreType.DMA,
    ],
)

BS = 1000


def _mlp_body(xg_ref, xo_ref, w_ref, b_ref, o_ref):
    xg = xg_ref[...]
    xo = xo_ref[...]
    w = w_ref[...]
    h = lax.dot_general(xg, w[:, :D], (((1,), (1,)), ((), ())),
                        preferred_element_type=jnp.float32)
    h = h + lax.dot_general(xo, w[:, D:], (((1,), (1,)), ((), ())),
                            preferred_element_type=jnp.float32)
    h = h + b_ref[...]
    h = jnp.where(h >= 0, h, 0.01 * h)
    o_ref[...] = h + xo


def _mlp(xg, xg_old, W, b2):
    return pl.pallas_call(
        _mlp_body,
        grid=(S // BS,),
        in_specs=[
            pl.BlockSpec((BS, D), lambda i: (i, 0)),
            pl.BlockSpec((BS, D), lambda i: (i, 0)),
            pl.BlockSpec((D, 2 * D), lambda i: (0, 0)),
            pl.BlockSpec((1, D), lambda i: (0, 0)),
        ],
        out_specs=pl.BlockSpec((BS, D), lambda i: (i, 0)),
        out_shape=jax.ShapeDtypeStruct((S, D), jnp.float32),
    )(xg, xg_old, W, b2)


def kernel(xg_old, x, batch, W, b):
    batch = batch.astype(jnp.int32)
    out_ref = jax.new_ref(jnp.full((S_PAD, D), NEG_INF, dtype=jnp.float32))
    bndr, bnds = _seg_call(x, batch, out_ref)
    _comb_call(bndr, bnds, out_ref)
    xg = out_ref[...][:S]
    return _mlp(xg, xg_old, W, b.reshape(1, D))


# speedup vs baseline: 1.7130x; 1.0148x over previous
import jax
import jax.numpy as jnp
from jax import lax
from jax.experimental import pallas as pl
from jax.experimental.pallas import tpu as pltpu
from jax.experimental.pallas import tpu_sc as plsc

N = 320000
S = 10000
D = 128
DUMMY = S
S_PAD = S + 8
NC = 2
NS = 16
NW = NC * NS
RPW = N // NW
CH = 400
NCH = RPW // CH
NG = CH // 16
CO = 128
NV = D // 16
NB = 2 * NW
NEG_INF = float("-inf")

_mesh = plsc.VectorSubcoreMesh(
    core_axis_name="c", subcore_axis_name="s", num_cores=NC, num_subcores=NS
)


def _seg_body(x_hbm, b_hbm, out_hbm, bndr_hbm, bnds_hbm,
              xb0, xb1, bb0, bb1, stage, sidx, sidxp, bndr_v, bnds_v, accb,
              semx0, semx1, semb0, semb1, sems, semo):
    wid = lax.axis_index("s") * NC + lax.axis_index("c")
    r0 = wid * RPW

    def issue(k, xb, bb, semx, semb):
        base = r0 + k * CH
        pltpu.async_copy(x_hbm.at[pl.ds(base, CH)], xb, semx)
        pltpu.async_copy(b_hbm.at[pl.ds(base, CH)], bb.at[pl.ds(0, CH)], semb)

    def wait(k, xb, bb, semx, semb):
        base = r0 + k * CH
        pltpu.make_async_copy(x_hbm.at[pl.ds(base, CH)], xb, semx).wait()
        pltpu.make_async_copy(
            b_hbm.at[pl.ds(base, CH)], bb.at[pl.ds(0, CH)], semb
        ).wait()

    issue(0, xb0, bb0, semx0, semb0)
    issue(1, xb1, bb1, semx1, semb1)

    def do_flush():
        for v in range(CO // 16):
            sidx[pl.ds(16 * v, 16)] = sidxp[pl.ds(16 * v, 16)]
        pltpu.async_copy(stage, out_hbm.at[sidx], sems).wait()

    def run_chunk(xb, bb, carry):
        def group(g, c):
            bvec = bb[pl.ds(16 * g, 16)]
            first, last = bvec[0], bvec[15]
            clean = jnp.logical_and(first == c[0], first == last)

            def fast(c):
                acc = [accb[pl.ds(16 * j, 16)] for j in range(NV)]
                for i in range(16):
                    for j in range(NV):
                        acc[j] = jnp.maximum(
                            acc[j], xb[16 * g + i, pl.ds(16 * j, 16)]
                        )
                for j in range(NV):
                    accb[pl.ds(16 * j, 16)] = acc[j]
                return c

            def slow(c):
                acc = [accb[pl.ds(16 * j, 16)] for j in range(NV)]
                for i in range(16):
                    cur, cnt, fd = c
                    s = bvec[i]
                    ch = s != cur
                    acc_now = list(acc)

                    def on_change(cnt_, fd_):
                        def real(cnt__, fd__):
                            def first_fn(cnt3):
                                for j in range(NV):
                                    bndr_v[0, pl.ds(16 * j, 16)] = acc_now[j]
                                bnds_v[0, :] = jnp.broadcast_to(cur, (16,))
                                return cnt3

                            def interior(cnt3):
                                for j in range(NV):
                                    stage[cnt3, pl.ds(16 * j, 16)] = acc_now[j]
                                sidxp[pl.ds(cnt3, 16)] = jnp.broadcast_to(
                                    cur, (16,)
                                )
                                cnt4 = cnt3 + 1

                                def flush():
                                    do_flush()
                                    return jnp.int32(0)

                                return lax.cond(
                                    cnt4 == CO, flush, lambda: cnt4
                                )

                            cnt_o = lax.cond(
                                fd__ == 1, interior, first_fn, cnt__
                            )
                            return cnt_o, jnp.int32(1)

                        return lax.cond(
                            cur >= 0, real, lambda a, b_: (a, b_), cnt_, fd_
                        )

                    cnt_n, fd_n = lax.cond(
                        ch, on_change, lambda a, b_: (a, b_), cnt, fd
                    )
                    for j in range(NV):
                        xv = xb[16 * g + i, pl.ds(16 * j, 16)]
                        acc[j] = jnp.where(
                            ch, xv, jnp.maximum(acc_now[j], xv)
                        )
                    c = (s, cnt_n, fd_n)
                for j in range(NV):
                    accb[pl.ds(16 * j, 16)] = acc[j]
                return c

            return lax.cond(clean, fast, slow, c)

        return lax.fori_loop(0, NG, group, carry)

    neg = jnp.full((16,), NEG_INF, jnp.float32)
    for j in range(NV):
        accb[pl.ds(16 * j, 16)] = neg
    carry0 = (jnp.int32(-1), jnp.int32(0), jnp.int32(0))
    npair = NCH // 2

    def pair(p, carry):
        wait(2 * p, xb0, bb0, semx0, semb0)
        carry = run_chunk(xb0, bb0, carry)
        issue(2 * p + 2, xb0, bb0, semx0, semb0)

        wait(2 * p + 1, xb1, bb1, semx1, semb1)
        carry = run_chunk(xb1, bb1, carry)

        @pl.when(p < npair - 1)
        def _():
            issue(2 * p + 3, xb1, bb1, semx1, semb1)

        return carry

    carry = lax.fori_loop(0, npair, pair, carry0)
    wait(NCH - 1, xb0, bb0, semx0, semb0)
    carry = run_chunk(xb0, bb0, carry)

    cur, cnt, fd = carry[0], carry[1], carry[2]
    acc = [accb[pl.ds(16 * j, 16)] for j in range(NV)]

    for j in range(NV):
        bndr_v[1, pl.ds(16 * j, 16)] = acc[j]
    bnds_v[1, :] = jnp.broadcast_to(cur, (16,))

    @pl.when(fd == 0)
    def _():
        for j in range(NV):
            bndr_v[0, pl.ds(16 * j, 16)] = acc[j]
        bnds_v[0, :] = jnp.broadcast_to(cur, (16,))

    lanes = lax.iota(jnp.int32, 16)
    for v in range(CO // 16):
        old = sidxp[pl.ds(16 * v, 16)]
        sidxp[pl.ds(16 * v, 16)] = jnp.where(
            lanes + 16 * v >= cnt, jnp.int32(DUMMY), old
        )
    do_flush()

    pltpu.async_copy(bndr_v, bndr_hbm.at[pl.ds(2 * wid, 2)], semo).wait()
    pltpu.async_copy(bnds_v, bnds_hbm.at[pl.ds(2 * wid, 2)], semo).wait()


_seg_call = pl.kernel(
    _seg_body,
    out_type=(
        jax.ShapeDtypeStruct((NB, D), jnp.float32),
        jax.ShapeDtypeStruct((NB, 16), jnp.int32),
    ),
    mesh=_mesh,
    compiler_params=pltpu.CompilerParams(
        use_tc_tiling_on_sc=False, needs_layout_passes=False
    ),
    scratch_types=[
        pltpu.VMEM((CH, D), jnp.float32),
        pltpu.VMEM((CH, D), jnp.float32),
        pltpu.VMEM((CH + 16,), jnp.int32),
        pltpu.VMEM((CH + 16,), jnp.int32),
        pltpu.VMEM((CO, D), jnp.float32),
        pltpu.VMEM((CO,), jnp.int32),
        pltpu.VMEM((CO + 16,), jnp.int32),
        pltpu.VMEM((2, D), jnp.float32),
        pltpu.VMEM((2, 16), jnp.int32),
        pltpu.VMEM((D,), jnp.float32),
        pltpu.SemaphoreType.DMA,
        pltpu.SemaphoreType.DMA,
        pltpu.SemaphoreType.DMA,
        pltpu.SemaphoreType.DMA,
        pltpu.SemaphoreType.DMA,
        pltpu.SemaphoreType.DMA,
    ],
)


def _comb_body(bndr_hbm, bnds_hbm, out_hbm, br, bs, stage2, sidx2, sidx2p,
               sem1, sem2):
    wid = lax.axis_index("s") * NC + lax.axis_index("c")

    @pl.when(wid == 0)
    def _():
        pltpu.async_copy(bndr_hbm, br, sem1).wait()
        pltpu.async_copy(bnds_hbm, bs, sem1).wait()

        acc0 = tuple(br[0, pl.ds(16 * j, 16)] for j in range(NV))
        carry0 = (bs[0, pl.ds(0, 16)][0], jnp.int32(0)) + acc0

        def rec(r, c):
            cur, cnt = c[0], c[1]
            acc = c[2:]
            s = bs[r, pl.ds(0, 16)][0]
            ch = s != cur

            def close(cnt_):
                for j in range(NV):
                    stage2[cnt_, pl.ds(16 * j, 16)] = acc[j]
                sidx2p[pl.ds(cnt_, 16)] = jnp.broadcast_to(cur, (16,))
                return cnt_ + 1

            cnt_n = lax.cond(ch, close, lambda a: a, cnt)
            rv = [br[r, pl.ds(16 * j, 16)] for j in range(NV)]
            acc_n = [
                jnp.where(ch, rv[j], jnp.maximum(acc[j], rv[j]))
                for j in range(NV)
            ]
            return (s, cnt_n) + tuple(acc_n)

        carry = lax.fori_loop(1, NB, rec, carry0)
        cur, cnt = carry[0], carry[1]
        acc = carry[2:]
        for j in range(NV):
            stage2[cnt, pl.ds(16 * j, 16)] = acc[j]
        sidx2p[pl.ds(cnt, 16)] = jnp.broadcast_to(cur, (16,))
        cnt = cnt + 1

        lanes = lax.iota(jnp.int32, 16)
        for v in range(NB // 16):
            old = sidx2p[pl.ds(16 * v, 16)]
            sidx2p[pl.ds(16 * v, 16)] = jnp.where(
                lanes + 16 * v >= cnt, jnp.int32(DUMMY), old
            )
            sidx2[pl.ds(16 * v, 16)] = sidx2p[pl.ds(16 * v, 16)]
        pltpu.async_copy(stage2, out_hbm.at[sidx2], sem2).wait()


_comb_call = pl.kernel(
    _comb_body,
    out_type=(),
    mesh=_mesh,
    compiler_params=pltpu.CompilerParams(
        use_tc_tiling_on_sc=False, needs_layout_passes=False
    ),
    scratch_types=[
        pltpu.VMEM((NB, D), jnp.float32),
        pltpu.VMEM((NB, 16), jnp.int32),
        pltpu.VMEM((NB, D), jnp.float32),
        pltpu.VMEM((NB,), jnp.int32),
        pltpu.VMEM((NB + 16,), jnp.int32),
        pltpu.SemaphoreType.DMA,
        pltpu.SemaphoreType.DMA,
    ],
)

BS = 1000


def _mlp_body(xg_ref, xo_ref, w_ref, b_ref, o_ref):
    xg = xg_ref[...]
    xo = xo_ref[...]
    w = w_ref[...]
    h = lax.dot_general(xg, w[:, :D], (((1,), (1,)), ((), ())),
                        preferred_element_type=jnp.float32)
    h = h + lax.dot_general(xo, w[:, D:], (((1,), (1,)), ((), ())),
                            preferred_element_type=jnp.float32)
    h = h + b_ref[...]
    h = jnp.where(h >= 0, h, 0.01 * h)
    o_ref[...] = h + xo


def _mlp(xg_pad, xg_old, W, b2):
    return pl.pallas_call(
        _mlp_body,
        grid=(S // BS,),
        in_specs=[
            pl.BlockSpec((BS, D), lambda i: (i, 0)),
            pl.BlockSpec((BS, D), lambda i: (i, 0)),
            pl.BlockSpec((D, 2 * D), lambda i: (0, 0)),
            pl.BlockSpec((1, D), lambda i: (0, 0)),
        ],
        out_specs=pl.BlockSpec((BS, D), lambda i: (i, 0)),
        out_shape=jax.ShapeDtypeStruct((S, D), jnp.float32),
    )(xg_pad, xg_old, W, b2)


def kernel(xg_old, x, batch, W, b):
    batch = batch.astype(jnp.int32)
    out_ref = jax.new_ref(jnp.full((S_PAD, D), NEG_INF, dtype=jnp.float32))
    bndr, bnds = _seg_call(x, batch, out_ref)
    _comb_call(bndr, bnds, out_ref)
    return _mlp(out_ref[...], xg_old, W, b.reshape(1, D))
